# baseline (speedup 1.0000x reference)
import jax
import jax.numpy as jnp
from jax.experimental import pallas as pl
from jax.experimental.pallas import tpu as pltpu

_ROWS = 8192
_COLS = 2048
_BLK = 512
_GRID = _ROWS // _BLK


def _body(x_ref, y_ref, m0_ref, m1_ref, o_ref, acc_ref):
    i = pl.program_id(0)

    @pl.when(i == 0)
    def _init():
        acc_ref[0] = 0.0
        acc_ref[1] = 0.0
        acc_ref[2] = 0.0
        acc_ref[3] = 0.0

    d = jnp.abs(x_ref[...] - y_ref[...])
    w0 = 1.0 - m0_ref[...].astype(jnp.float32)
    w1 = 1.0 - m1_ref[...].astype(jnp.float32)
    acc_ref[0] += jnp.sum(d * w0)
    acc_ref[1] += jnp.sum(w0)
    acc_ref[2] += jnp.sum(d * w1)
    acc_ref[3] += jnp.sum(w1)

    @pl.when(i == _GRID - 1)
    def _fin():
        o_ref[0] = acc_ref[0] / acc_ref[1] + acc_ref[2] / acc_ref[3]


def kernel(input, output, mask0, mask1):
    x = input.reshape(_ROWS, _COLS)
    y = output.reshape(_ROWS, _COLS)
    m0 = mask0.reshape(_ROWS, _COLS)
    m1 = mask1.reshape(_ROWS, _COLS)

    spec = pl.BlockSpec((_BLK, _COLS), lambda i: (i, 0))
    out = pl.pallas_call(
        _body,
        grid=(_GRID,),
        in_specs=[spec, spec, spec, spec],
        out_specs=pl.BlockSpec(memory_space=pltpu.SMEM),
        out_shape=jax.ShapeDtypeStruct((1,), jnp.float32),
        scratch_shapes=[pltpu.SMEM((4,), jnp.float32)],
        compiler_params=pltpu.CompilerParams(
            dimension_semantics=("arbitrary",),
        ),
    )(x, y, m0, m1)
    return out[0]
